# baseline (speedup 1.0000x reference)
import functools

import jax
import jax.numpy as jnp
from jax import lax
from jax.experimental import pallas as pl
from jax.experimental.pallas import tpu as pltpu
from jax.experimental.pallas import tpu_sc as plsc

N = 10000
E = 320000
D = 128
H = 128
O = 64

NC = 2
NS = 16
NW = NC * NS
CE = 128
NCH = 80
EPT = NCH * CE
EPAD = NW * EPT
NPAD = 10240
RPS = NPAD // NS

_mesh = plsc.VectorSubcoreMesh(core_axis_name="c", subcore_axis_name="s")


@functools.partial(
    pl.kernel,
    out_type=jax.ShapeDtypeStruct((NC, NPAD), jnp.float32),
    mesh=_mesh,
    scratch_types=[
        pltpu.VMEM((NCH, CE), jnp.int32),
        pltpu.VMEM((CE,), jnp.float32),
        pltpu.VMEM_SHARED((NPAD,), jnp.float32),
        pltpu.SemaphoreType.DMA,
    ],
)
def _degree(dsts_hbm, ones_hbm, zeros1_hbm, out_hbm, dst_v, ones_v, cnt_sh, sem):
    c = lax.axis_index("c")
    s = lax.axis_index("s")
    wid = s * NC + c
    pltpu.async_copy(dsts_hbm.at[wid], dst_v, sem).wait()
    pltpu.async_copy(ones_hbm, ones_v, sem).wait()
    pltpu.async_copy(
        zeros1_hbm.at[pl.ds(s * RPS, RPS)], cnt_sh.at[pl.ds(s * RPS, RPS)], sem
    ).wait()
    plsc.subcore_barrier()

    @pl.loop(0, NCH)
    def _(j):
        pltpu.sync_copy(ones_v, cnt_sh.at[dst_v.at[j]], add=True)

    plsc.subcore_barrier()
    pltpu.async_copy(
        cnt_sh.at[pl.ds(s * RPS, RPS)], out_hbm.at[c].at[pl.ds(s * RPS, RPS)], sem
    ).wait()


def _make_propagate(width):
    @functools.partial(
        pl.kernel,
        out_type=jax.ShapeDtypeStruct((NC, NPAD, width), jnp.float32),
        mesh=_mesh,
        compiler_params=pltpu.CompilerParams(use_tc_tiling_on_sc=False),
        scratch_types=[
            pltpu.VMEM((NCH // 2, CE), jnp.int32),
            pltpu.VMEM((NCH // 2, CE), jnp.int32),
            pltpu.VMEM((CE, width), jnp.float32),
            pltpu.VMEM((CE, width), jnp.float32),
            pltpu.VMEM_SHARED((NPAD, width), jnp.float32),
            pltpu.SemaphoreType.DMA,
            pltpu.SemaphoreType.DMA,
            pltpu.SemaphoreType.DMA,
            pltpu.SemaphoreType.DMA,
            pltpu.SemaphoreType.DMA,
        ],
    )
    def prop(g_hbm, srcs_hbm, dsts_hbm, zeros_hbm, out_hbm,
             src_v, dst_v, buf0, buf1, acc_sh, sem, gsem0, gsem1, ssem0, ssem1):
        c = lax.axis_index("c")
        s = lax.axis_index("s")
        wid = s * NC + c
        hch = NCH // 2
        pltpu.async_copy(
            zeros_hbm.at[pl.ds(s * RPS, RPS)], acc_sh.at[pl.ds(s * RPS, RPS)], sem
        ).wait()
        plsc.subcore_barrier()

        @pl.loop(0, 2)
        def _(p):
            cp_s = pltpu.async_copy(srcs_hbm.at[wid].at[pl.ds(p * hch, hch)],
                                    src_v, sem)
            cp_d = pltpu.async_copy(dsts_hbm.at[wid].at[pl.ds(p * hch, hch)],
                                    dst_v, gsem0)
            cp_s.wait()
            cp_d.wait()
            pltpu.async_copy(g_hbm.at[src_v.at[0]], buf0, gsem0)
            pltpu.async_copy(g_hbm.at[src_v.at[1]], buf1, gsem1)

            @pl.loop(0, hch, step=2)
            def _(j):
                pltpu.make_async_copy(g_hbm.at[src_v.at[j]], buf0, gsem0).wait()
                pltpu.async_copy(buf0, acc_sh.at[dst_v.at[j]], ssem0, add=True)
                pltpu.make_async_copy(g_hbm.at[src_v.at[j + 1]], buf1, gsem1).wait()
                pltpu.async_copy(buf1, acc_sh.at[dst_v.at[j + 1]], ssem1, add=True)

                pltpu.make_async_copy(buf0, acc_sh.at[dst_v.at[j]], ssem0).wait()

                @pl.when(j + 2 < hch)
                def _():
                    pltpu.async_copy(g_hbm.at[src_v.at[j + 2]], buf0, gsem0)

                pltpu.make_async_copy(buf1, acc_sh.at[dst_v.at[j + 1]], ssem1).wait()

                @pl.when(j + 3 < hch)
                def _():
                    pltpu.async_copy(g_hbm.at[src_v.at[j + 3]], buf1, gsem1)

        plsc.subcore_barrier()
        pltpu.async_copy(
            acc_sh.at[pl.ds(s * RPS, RPS)],
            out_hbm.at[c].at[pl.ds(s * RPS, RPS)],
            sem,
        ).wait()

    return prop


_prop_h = _make_propagate(H)
_prop_o = _make_propagate(O)


_BLK = 1280
_GRID = NPAD // _BLK


def _matmul(x, w):
    d_in, d_out = w.shape

    def body(x_ref, w_ref, o_ref):
        o_ref[...] = jnp.dot(x_ref[...], w_ref[...],
                             preferred_element_type=jnp.float32)

    return pl.pallas_call(
        body,
        grid=(_GRID,),
        in_specs=[
            pl.BlockSpec((_BLK, d_in), lambda i: (i, 0)),
            pl.BlockSpec((d_in, d_out), lambda i: (0, 0)),
        ],
        out_specs=pl.BlockSpec((_BLK, d_out), lambda i: (i, 0)),
        out_shape=jax.ShapeDtypeStruct((NPAD, d_out), jnp.float32),
    )(x, w)


def _scale(h, counts):
    def body(h_ref, c_ref, o_ref):
        dinv = lax.rsqrt(1.0 + c_ref[0, :] + c_ref[1, :])
        o_ref[...] = h_ref[...] * dinv[:, None]

    return pl.pallas_call(
        body,
        grid=(_GRID,),
        in_specs=[
            pl.BlockSpec((_BLK, H), lambda i: (i, 0)),
            pl.BlockSpec((NC, _BLK), lambda i: (0, i)),
        ],
        out_specs=pl.BlockSpec((_BLK, H), lambda i: (i, 0)),
        out_shape=jax.ShapeDtypeStruct((NPAD, H), jnp.float32),
    )(h, counts)


def _layer2(acc1, g1, counts, b1, w2):
    def body(a_ref, g_ref, c_ref, b_ref, w_ref, h_ref, g2_ref):
        dinv = lax.rsqrt(1.0 + c_ref[0, :] + c_ref[1, :])
        pre = (a_ref[0] + a_ref[1] + g_ref[...]) * dinv[:, None] + b_ref[...]
        hblk = jnp.maximum(pre, 0.0)
        h_ref[...] = hblk
        g2_ref[...] = jnp.dot(hblk, w_ref[...],
                              preferred_element_type=jnp.float32) * dinv[:, None]

    return pl.pallas_call(
        body,
        grid=(_GRID,),
        in_specs=[
            pl.BlockSpec((NC, _BLK, H), lambda i: (0, i, 0)),
            pl.BlockSpec((_BLK, H), lambda i: (i, 0)),
            pl.BlockSpec((NC, _BLK), lambda i: (0, i)),
            pl.BlockSpec((1, H), lambda i: (0, 0)),
            pl.BlockSpec((H, O), lambda i: (0, 0)),
        ],
        out_specs=[
            pl.BlockSpec((_BLK, H), lambda i: (i, 0)),
            pl.BlockSpec((_BLK, O), lambda i: (i, 0)),
        ],
        out_shape=[
            jax.ShapeDtypeStruct((NPAD, H), jnp.float32),
            jax.ShapeDtypeStruct((NPAD, O), jnp.float32),
        ],
    )(acc1, g1, counts, b1, w2)


def _final(acc2, g2, counts, b2):
    def body(a_ref, g_ref, c_ref, b_ref, o_ref):
        dinv = lax.rsqrt(1.0 + c_ref[0, :] + c_ref[1, :])
        o_ref[...] = (a_ref[0] + a_ref[1] + g_ref[...]) * dinv[:, None] + b_ref[...]

    return pl.pallas_call(
        body,
        grid=(_GRID,),
        in_specs=[
            pl.BlockSpec((NC, _BLK, O), lambda i: (0, i, 0)),
            pl.BlockSpec((_BLK, O), lambda i: (i, 0)),
            pl.BlockSpec((NC, _BLK), lambda i: (0, i)),
            pl.BlockSpec((1, O), lambda i: (0, 0)),
        ],
        out_specs=pl.BlockSpec((_BLK, O), lambda i: (i, 0)),
        out_shape=jax.ShapeDtypeStruct((NPAD, O), jnp.float32),
    )(acc2, g2, counts, b2)


def kernel(x, edge_index, W1, b1, W2, b2):
    src = edge_index[0].astype(jnp.int32)
    dst = edge_index[1].astype(jnp.int32)
    pad_src = N + jnp.arange(EPAD - E, dtype=jnp.int32) % (NPAD - N)
    pad_dst = pad_src
    srcs = jnp.concatenate([src, pad_src]).reshape(NW, NCH, CE)
    dsts = jnp.concatenate([dst, pad_dst]).reshape(NW, NCH, CE)

    x_pad = jnp.zeros((NPAD, D), jnp.float32).at[:N].set(x)
    ones_c = jnp.ones((CE,), jnp.float32)
    zeros1 = jnp.zeros((NPAD,), jnp.float32)
    zeros_h = jnp.zeros((NPAD, H), jnp.float32)
    zeros_o = jnp.zeros((NPAD, O), jnp.float32)

    counts = _degree(dsts, ones_c, zeros1)
    h1 = _matmul(x_pad, W1)
    g1 = _scale(h1, counts)
    acc1 = _prop_h(g1, srcs, dsts, zeros_h)
    h_pad, g2 = _layer2(acc1, g1, counts, b1.reshape(1, H), W2)
    acc2 = _prop_o(g2, srcs, dsts, zeros_o)
    logits_pad = _final(acc2, g2, counts, b2.reshape(1, O))
    return h_pad[:N], logits_pad[:N]


# speedup vs baseline: 1.1884x; 1.1884x over previous
import functools

import jax
import jax.numpy as jnp
from jax import lax
from jax.experimental import pallas as pl
from jax.experimental.pallas import tpu as pltpu
from jax.experimental.pallas import tpu_sc as plsc

N = 10000
E = 320000
D = 128
H = 128
O = 64

NC = 2
NS = 16
NW = NC * NS
CE = 128
NCH = 80
RE = E // CE
TAIL = RE - (NW - 1) * NCH
PAD_CH = NW * NCH - RE
NPAD = 10240
RPS = NPAD // NS

_mesh = plsc.VectorSubcoreMesh(core_axis_name="c", subcore_axis_name="s")


@functools.partial(
    pl.kernel,
    out_type=jax.ShapeDtypeStruct((NC, NPAD), jnp.float32),
    mesh=_mesh,
    scratch_types=[
        pltpu.VMEM((NCH, CE), jnp.int32),
        pltpu.VMEM((CE,), jnp.float32),
        pltpu.VMEM_SHARED((NPAD,), jnp.float32),
        pltpu.SemaphoreType.DMA,
    ],
)
def _degree(dsts_hbm, pads_hbm, ones_hbm, zeros1_hbm, out_hbm,
            dst_v, ones_v, cnt_sh, sem):
    c = lax.axis_index("c")
    s = lax.axis_index("s")
    wid = s * NC + c

    @pl.when(wid < NW - 1)
    def _():
        pltpu.async_copy(dsts_hbm.at[pl.ds(wid * NCH, NCH)], dst_v, sem).wait()

    @pl.when(wid == NW - 1)
    def _():
        cp_a = pltpu.async_copy(dsts_hbm.at[pl.ds(RE - TAIL, TAIL)],
                                dst_v.at[pl.ds(0, TAIL)], sem)
        cp_b = pltpu.async_copy(pads_hbm, dst_v.at[pl.ds(TAIL, PAD_CH)], sem)
        cp_a.wait()
        cp_b.wait()

    pltpu.async_copy(ones_hbm, ones_v, sem).wait()
    pltpu.async_copy(
        zeros1_hbm.at[pl.ds(s * RPS, RPS)], cnt_sh.at[pl.ds(s * RPS, RPS)], sem
    ).wait()
    plsc.subcore_barrier()

    @pl.loop(0, NCH)
    def _(j):
        pltpu.sync_copy(ones_v, cnt_sh.at[dst_v.at[j]], add=True)

    plsc.subcore_barrier()
    pltpu.async_copy(
        cnt_sh.at[pl.ds(s * RPS, RPS)], out_hbm.at[c].at[pl.ds(s * RPS, RPS)], sem
    ).wait()


def _make_propagate(width):
    @functools.partial(
        pl.kernel,
        out_type=jax.ShapeDtypeStruct((NC, NPAD, width), jnp.float32),
        mesh=_mesh,
        compiler_params=pltpu.CompilerParams(use_tc_tiling_on_sc=False),
        scratch_types=[
            pltpu.VMEM((NCH // 2, CE), jnp.int32),
            pltpu.VMEM((NCH // 2, CE), jnp.int32),
            pltpu.VMEM((CE, width), jnp.float32),
            pltpu.VMEM((CE, width), jnp.float32),
            pltpu.VMEM_SHARED((NPAD, width), jnp.float32),
            pltpu.SemaphoreType.DMA,
            pltpu.SemaphoreType.DMA,
            pltpu.SemaphoreType.DMA,
        ],
    )
    def prop(g_hbm, srcs_hbm, dsts_hbm, pads_hbm, zeros_hbm, out_hbm,
             src_v, dst_v, buf0, buf1, acc_sh, sem, gsem0, gsem1):
        c = lax.axis_index("c")
        s = lax.axis_index("s")
        wid = s * NC + c
        hch = NCH // 2
        pltpu.async_copy(
            zeros_hbm.at[pl.ds(s * RPS, RPS)], acc_sh.at[pl.ds(s * RPS, RPS)], sem
        ).wait()
        plsc.subcore_barrier()

        @pl.loop(0, 2)
        def _(p):
            row0 = wid * NCH + p * hch

            @pl.when(wid < NW - 1)
            def _():
                cp_s = pltpu.async_copy(srcs_hbm.at[pl.ds(row0, hch)], src_v, sem)
                cp_d = pltpu.async_copy(dsts_hbm.at[pl.ds(row0, hch)], dst_v, gsem0)
                cp_s.wait()
                cp_d.wait()

            @pl.when((wid == NW - 1) & (p == 0))
            def _():
                cp_a = pltpu.async_copy(srcs_hbm.at[pl.ds(RE - TAIL, TAIL)],
                                        src_v.at[pl.ds(0, TAIL)], sem)
                cp_b = pltpu.async_copy(pads_hbm.at[pl.ds(0, hch - TAIL)],
                                        src_v.at[pl.ds(TAIL, hch - TAIL)], gsem0)
                cp_c = pltpu.async_copy(dsts_hbm.at[pl.ds(RE - TAIL, TAIL)],
                                        dst_v.at[pl.ds(0, TAIL)], gsem1)
                cp_d = pltpu.async_copy(pads_hbm.at[pl.ds(0, hch - TAIL)],
                                        dst_v.at[pl.ds(TAIL, hch - TAIL)], sem)
                cp_a.wait()
                cp_b.wait()
                cp_c.wait()
                cp_d.wait()

            @pl.when((wid == NW - 1) & (p == 1))
            def _():
                cp_s = pltpu.async_copy(pads_hbm.at[pl.ds(PAD_CH - hch, hch)],
                                        src_v, sem)
                cp_d = pltpu.async_copy(pads_hbm.at[pl.ds(PAD_CH - hch, hch)],
                                        dst_v, gsem0)
                cp_s.wait()
                cp_d.wait()
            pltpu.async_copy(g_hbm.at[src_v.at[0]], buf0, gsem0)
            pltpu.async_copy(g_hbm.at[src_v.at[1]], buf1, gsem1)

            @pl.loop(0, hch, step=2)
            def _(j):
                pltpu.make_async_copy(g_hbm.at[src_v.at[j]], buf0, gsem0).wait()
                pltpu.sync_copy(buf0, acc_sh.at[dst_v.at[j]], add=True)

                @pl.when(j + 2 < hch)
                def _():
                    pltpu.async_copy(g_hbm.at[src_v.at[j + 2]], buf0, gsem0)

                pltpu.make_async_copy(g_hbm.at[src_v.at[j + 1]], buf1, gsem1).wait()
                pltpu.sync_copy(buf1, acc_sh.at[dst_v.at[j + 1]], add=True)

                @pl.when(j + 3 < hch)
                def _():
                    pltpu.async_copy(g_hbm.at[src_v.at[j + 3]], buf1, gsem1)

        plsc.subcore_barrier()
        pltpu.async_copy(
            acc_sh.at[pl.ds(s * RPS, RPS)],
            out_hbm.at[c].at[pl.ds(s * RPS, RPS)],
            sem,
        ).wait()

    return prop


_prop_h = _make_propagate(H)
_prop_o = _make_propagate(O)


_BLK = 1280
_GRID = NPAD // _BLK


def _matmul(x, w):
    d_in, d_out = w.shape

    def body(x_ref, w_ref, o_ref):
        o_ref[...] = jnp.dot(x_ref[...], w_ref[...],
                             preferred_element_type=jnp.float32)

    return pl.pallas_call(
        body,
        grid=(_GRID,),
        in_specs=[
            pl.BlockSpec((_BLK, d_in), lambda i: (i, 0)),
            pl.BlockSpec((d_in, d_out), lambda i: (0, 0)),
        ],
        out_specs=pl.BlockSpec((_BLK, d_out), lambda i: (i, 0)),
        out_shape=jax.ShapeDtypeStruct((NPAD, d_out), jnp.float32),
    )(x, w)


def _scale(h, counts):
    def body(h_ref, c_ref, o_ref):
        dinv = lax.rsqrt(1.0 + c_ref[0, :] + c_ref[1, :])
        o_ref[...] = h_ref[...] * dinv[:, None]

    return pl.pallas_call(
        body,
        grid=(_GRID,),
        in_specs=[
            pl.BlockSpec((_BLK, H), lambda i: (i, 0)),
            pl.BlockSpec((NC, _BLK), lambda i: (0, i)),
        ],
        out_specs=pl.BlockSpec((_BLK, H), lambda i: (i, 0)),
        out_shape=jax.ShapeDtypeStruct((NPAD, H), jnp.float32),
    )(h, counts)


def _layer2(acc1, g1, counts, b1, w2):
    def body(a_ref, g_ref, c_ref, b_ref, w_ref, h_ref, g2_ref):
        dinv = lax.rsqrt(1.0 + c_ref[0, :] + c_ref[1, :])
        pre = (a_ref[0] + a_ref[1] + g_ref[...]) * dinv[:, None] + b_ref[...]
        hblk = jnp.maximum(pre, 0.0)
        h_ref[...] = hblk
        g2_ref[...] = jnp.dot(hblk, w_ref[...],
                              preferred_element_type=jnp.float32) * dinv[:, None]

    return pl.pallas_call(
        body,
        grid=(_GRID,),
        in_specs=[
            pl.BlockSpec((NC, _BLK, H), lambda i: (0, i, 0)),
            pl.BlockSpec((_BLK, H), lambda i: (i, 0)),
            pl.BlockSpec((NC, _BLK), lambda i: (0, i)),
            pl.BlockSpec((1, H), lambda i: (0, 0)),
            pl.BlockSpec((H, O), lambda i: (0, 0)),
        ],
        out_specs=[
            pl.BlockSpec((_BLK, H), lambda i: (i, 0)),
            pl.BlockSpec((_BLK, O), lambda i: (i, 0)),
        ],
        out_shape=[
            jax.ShapeDtypeStruct((N, H), jnp.float32),
            jax.ShapeDtypeStruct((NPAD, O), jnp.float32),
        ],
    )(acc1, g1, counts, b1, w2)


def _final(acc2, g2, counts, b2):
    def body(a_ref, g_ref, c_ref, b_ref, o_ref):
        dinv = lax.rsqrt(1.0 + c_ref[0, :] + c_ref[1, :])
        o_ref[...] = (a_ref[0] + a_ref[1] + g_ref[...]) * dinv[:, None] + b_ref[...]

    return pl.pallas_call(
        body,
        grid=(_GRID,),
        in_specs=[
            pl.BlockSpec((NC, _BLK, O), lambda i: (0, i, 0)),
            pl.BlockSpec((_BLK, O), lambda i: (i, 0)),
            pl.BlockSpec((NC, _BLK), lambda i: (0, i)),
            pl.BlockSpec((1, O), lambda i: (0, 0)),
        ],
        out_specs=pl.BlockSpec((_BLK, O), lambda i: (i, 0)),
        out_shape=jax.ShapeDtypeStruct((N, O), jnp.float32),
    )(acc2, g2, counts, b2)


def kernel(x, edge_index, W1, b1, W2, b2):
    srcs = edge_index[0].astype(jnp.int32).reshape(RE, CE)
    dsts = edge_index[1].astype(jnp.int32).reshape(RE, CE)
    pads = (N + jnp.arange(PAD_CH * CE, dtype=jnp.int32) % (NPAD - N)
            ).reshape(PAD_CH, CE)
    ones_c = jnp.ones((CE,), jnp.float32)
    zeros1 = jnp.zeros((NPAD,), jnp.float32)
    zeros_h = jnp.zeros((NPAD, H), jnp.float32)
    zeros_o = jnp.zeros((NPAD, O), jnp.float32)

    counts = _degree(dsts, pads, ones_c, zeros1)
    h1 = _matmul(x, W1)
    g1 = _scale(h1, counts)
    acc1 = _prop_h(g1, srcs, dsts, pads, zeros_h)
    h, g2 = _layer2(acc1, g1, counts, b1.reshape(1, H), W2)
    acc2 = _prop_o(g2, srcs, dsts, pads, zeros_o)
    logits = _final(acc2, g2, counts, b2.reshape(1, O))
    return h, logits


# speedup vs baseline: 1.2539x; 1.0550x over previous
import functools

import jax
import jax.numpy as jnp
from jax import lax
from jax.experimental import pallas as pl
from jax.experimental.pallas import tpu as pltpu
from jax.experimental.pallas import tpu_sc as plsc

N = 10000
E = 320000
D = 128
H = 128
O = 64

NC = 2
NS = 16
NW = NC * NS
CE = 128
NCH = 80
RE = E // CE
TAIL = RE - (NW - 1) * NCH
PAD_CH = NW * NCH - RE
NPAD = 10240
RPS = NPAD // NS

_mesh = plsc.VectorSubcoreMesh(core_axis_name="c", subcore_axis_name="s")


@functools.partial(
    pl.kernel,
    out_type=jax.ShapeDtypeStruct((NC, NPAD), jnp.float32),
    mesh=_mesh,
    scratch_types=[
        pltpu.VMEM((NCH * CE,), jnp.int32),
        pltpu.VMEM((CE,), jnp.float32),
        pltpu.VMEM_SHARED((NPAD,), jnp.float32),
        pltpu.SemaphoreType.DMA,
    ],
)
def _degree(edge_hbm, pads_hbm, ones_hbm, zeros1_hbm, out_hbm,
            dst_v, ones_v, cnt_sh, sem):
    c = lax.axis_index("c")
    s = lax.axis_index("s")
    wid = s * NC + c

    @pl.when(wid < NW - 1)
    def _():
        pltpu.async_copy(edge_hbm.at[1].at[pl.ds(wid * (NCH * CE), NCH * CE)],
                         dst_v, sem).wait()

    @pl.when(wid == NW - 1)
    def _():
        cp_a = pltpu.async_copy(
            edge_hbm.at[1].at[pl.ds((RE - TAIL) * CE, TAIL * CE)],
            dst_v.at[pl.ds(0, TAIL * CE)], sem)
        cp_b = pltpu.async_copy(pads_hbm,
                                dst_v.at[pl.ds(TAIL * CE, PAD_CH * CE)], sem)
        cp_a.wait()
        cp_b.wait()

    pltpu.async_copy(ones_hbm, ones_v, sem).wait()
    pltpu.async_copy(
        zeros1_hbm.at[pl.ds(s * RPS, RPS)], cnt_sh.at[pl.ds(s * RPS, RPS)], sem
    ).wait()
    plsc.subcore_barrier()

    @pl.loop(0, NCH)
    def _(j):
        pltpu.sync_copy(ones_v, cnt_sh.at[dst_v.at[pl.ds(j * CE, CE)]], add=True)

    plsc.subcore_barrier()
    pltpu.async_copy(
        cnt_sh.at[pl.ds(s * RPS, RPS)], out_hbm.at[c].at[pl.ds(s * RPS, RPS)], sem
    ).wait()


def _make_propagate(width):
    @functools.partial(
        pl.kernel,
        out_type=jax.ShapeDtypeStruct((NC, NPAD, width), jnp.float32),
        mesh=_mesh,
        compiler_params=pltpu.CompilerParams(use_tc_tiling_on_sc=False),
        scratch_types=[
            pltpu.VMEM((NCH // 2 * CE,), jnp.int32),
            pltpu.VMEM((NCH // 2 * CE,), jnp.int32),
            pltpu.VMEM((CE, width), jnp.float32),
            pltpu.VMEM((CE, width), jnp.float32),
            pltpu.VMEM_SHARED((NPAD, width), jnp.float32),
            pltpu.SemaphoreType.DMA,
            pltpu.SemaphoreType.DMA,
            pltpu.SemaphoreType.DMA,
        ],
    )
    def prop(g_hbm, edge_hbm, pads_hbm, zeros_hbm, out_hbm,
             src_v, dst_v, buf0, buf1, acc_sh, sem, gsem0, gsem1):
        c = lax.axis_index("c")
        s = lax.axis_index("s")
        wid = s * NC + c
        hch = NCH // 2
        pltpu.async_copy(
            zeros_hbm.at[pl.ds(s * RPS, RPS)], acc_sh.at[pl.ds(s * RPS, RPS)], sem
        ).wait()
        plsc.subcore_barrier()

        @pl.loop(0, 2)
        def _(p):
            e0 = (wid * NCH + p * hch) * CE

            @pl.when(wid < NW - 1)
            def _():
                cp_s = pltpu.async_copy(edge_hbm.at[0].at[pl.ds(e0, hch * CE)],
                                        src_v, sem)
                cp_d = pltpu.async_copy(edge_hbm.at[1].at[pl.ds(e0, hch * CE)],
                                        dst_v, gsem0)
                cp_s.wait()
                cp_d.wait()

            @pl.when((wid == NW - 1) & (p == 0))
            def _():
                cp_a = pltpu.async_copy(
                    edge_hbm.at[0].at[pl.ds((RE - TAIL) * CE, TAIL * CE)],
                    src_v.at[pl.ds(0, TAIL * CE)], sem)
                cp_b = pltpu.async_copy(
                    pads_hbm.at[pl.ds(0, (hch - TAIL) * CE)],
                    src_v.at[pl.ds(TAIL * CE, (hch - TAIL) * CE)], gsem0)
                cp_c = pltpu.async_copy(
                    edge_hbm.at[1].at[pl.ds((RE - TAIL) * CE, TAIL * CE)],
                    dst_v.at[pl.ds(0, TAIL * CE)], gsem1)
                cp_d = pltpu.async_copy(
                    pads_hbm.at[pl.ds(0, (hch - TAIL) * CE)],
                    dst_v.at[pl.ds(TAIL * CE, (hch - TAIL) * CE)], sem)
                cp_a.wait()
                cp_b.wait()
                cp_c.wait()
                cp_d.wait()

            @pl.when((wid == NW - 1) & (p == 1))
            def _():
                cp_s = pltpu.async_copy(
                    pads_hbm.at[pl.ds((PAD_CH - hch) * CE, hch * CE)], src_v, sem)
                cp_d = pltpu.async_copy(
                    pads_hbm.at[pl.ds((PAD_CH - hch) * CE, hch * CE)], dst_v, gsem0)
                cp_s.wait()
                cp_d.wait()
            pltpu.async_copy(g_hbm.at[src_v.at[pl.ds(0, CE)]], buf0, gsem0)
            pltpu.async_copy(g_hbm.at[src_v.at[pl.ds(CE, CE)]], buf1, gsem1)

            @pl.loop(0, hch, step=2)
            def _(j):
                pltpu.make_async_copy(
                    g_hbm.at[src_v.at[pl.ds(j * CE, CE)]], buf0, gsem0).wait()
                pltpu.sync_copy(buf0, acc_sh.at[dst_v.at[pl.ds(j * CE, CE)]],
                                add=True)

                @pl.when(j + 2 < hch)
                def _():
                    pltpu.async_copy(
                        g_hbm.at[src_v.at[pl.ds((j + 2) * CE, CE)]], buf0, gsem0)

                pltpu.make_async_copy(
                    g_hbm.at[src_v.at[pl.ds((j + 1) * CE, CE)]], buf1, gsem1).wait()
                pltpu.sync_copy(buf1, acc_sh.at[dst_v.at[pl.ds((j + 1) * CE, CE)]],
                                add=True)

                @pl.when(j + 3 < hch)
                def _():
                    pltpu.async_copy(
                        g_hbm.at[src_v.at[pl.ds((j + 3) * CE, CE)]], buf1, gsem1)

        plsc.subcore_barrier()
        pltpu.async_copy(
            acc_sh.at[pl.ds(s * RPS, RPS)],
            out_hbm.at[c].at[pl.ds(s * RPS, RPS)],
            sem,
        ).wait()

    return prop


_prop_h = _make_propagate(H)
_prop_o = _make_propagate(O)


_BLK = 1280
_GRID = NPAD // _BLK


def _matmul(x, w):
    d_in, d_out = w.shape

    def body(x_ref, w_ref, o_ref):
        o_ref[...] = jnp.dot(x_ref[...], w_ref[...],
                             preferred_element_type=jnp.float32)

    return pl.pallas_call(
        body,
        grid=(_GRID,),
        in_specs=[
            pl.BlockSpec((_BLK, d_in), lambda i: (i, 0)),
            pl.BlockSpec((d_in, d_out), lambda i: (0, 0)),
        ],
        out_specs=pl.BlockSpec((_BLK, d_out), lambda i: (i, 0)),
        out_shape=jax.ShapeDtypeStruct((NPAD, d_out), jnp.float32),
    )(x, w)


def _scale(h, counts):
    def body(h_ref, c_ref, o_ref):
        dinv = lax.rsqrt(1.0 + c_ref[0, :] + c_ref[1, :])
        o_ref[...] = h_ref[...] * dinv[:, None]

    return pl.pallas_call(
        body,
        grid=(_GRID,),
        in_specs=[
            pl.BlockSpec((_BLK, H), lambda i: (i, 0)),
            pl.BlockSpec((NC, _BLK), lambda i: (0, i)),
        ],
        out_specs=pl.BlockSpec((_BLK, H), lambda i: (i, 0)),
        out_shape=jax.ShapeDtypeStruct((NPAD, H), jnp.float32),
    )(h, counts)


def _layer2(acc1, g1, counts, b1, w2):
    def body(a_ref, g_ref, c_ref, b_ref, w_ref, h_ref, g2_ref):
        dinv = lax.rsqrt(1.0 + c_ref[0, :] + c_ref[1, :])
        pre = (a_ref[0] + a_ref[1] + g_ref[...]) * dinv[:, None] + b_ref[...]
        hblk = jnp.maximum(pre, 0.0)
        h_ref[...] = hblk
        g2_ref[...] = jnp.dot(hblk, w_ref[...],
                              preferred_element_type=jnp.float32) * dinv[:, None]

    return pl.pallas_call(
        body,
        grid=(_GRID,),
        in_specs=[
            pl.BlockSpec((NC, _BLK, H), lambda i: (0, i, 0)),
            pl.BlockSpec((_BLK, H), lambda i: (i, 0)),
            pl.BlockSpec((NC, _BLK), lambda i: (0, i)),
            pl.BlockSpec((1, H), lambda i: (0, 0)),
            pl.BlockSpec((H, O), lambda i: (0, 0)),
        ],
        out_specs=[
            pl.BlockSpec((_BLK, H), lambda i: (i, 0)),
            pl.BlockSpec((_BLK, O), lambda i: (i, 0)),
        ],
        out_shape=[
            jax.ShapeDtypeStruct((N, H), jnp.float32),
            jax.ShapeDtypeStruct((NPAD, O), jnp.float32),
        ],
    )(acc1, g1, counts, b1, w2)


def _final(acc2, g2, counts, b2):
    def body(a_ref, g_ref, c_ref, b_ref, o_ref):
        dinv = lax.rsqrt(1.0 + c_ref[0, :] + c_ref[1, :])
        o_ref[...] = (a_ref[0] + a_ref[1] + g_ref[...]) * dinv[:, None] + b_ref[...]

    return pl.pallas_call(
        body,
        grid=(_GRID,),
        in_specs=[
            pl.BlockSpec((NC, _BLK, O), lambda i: (0, i, 0)),
            pl.BlockSpec((_BLK, O), lambda i: (i, 0)),
            pl.BlockSpec((NC, _BLK), lambda i: (0, i)),
            pl.BlockSpec((1, O), lambda i: (0, 0)),
        ],
        out_specs=pl.BlockSpec((_BLK, O), lambda i: (i, 0)),
        out_shape=jax.ShapeDtypeStruct((N, O), jnp.float32),
    )(acc2, g2, counts, b2)


def kernel(x, edge_index, W1, b1, W2, b2):
    edges = edge_index.astype(jnp.int32)
    pads = N + jnp.arange(PAD_CH * CE, dtype=jnp.int32) % (NPAD - N)
    ones_c = jnp.ones((CE,), jnp.float32)
    zeros1 = jnp.zeros((NPAD,), jnp.float32)
    zeros_h = jnp.zeros((NPAD, H), jnp.float32)
    zeros_o = jnp.zeros((NPAD, O), jnp.float32)

    counts = _degree(edges, pads, ones_c, zeros1)
    h1 = _matmul(x, W1)
    g1 = _scale(h1, counts)
    acc1 = _prop_h(g1, edges, pads, zeros_h)
    h, g2 = _layer2(acc1, g1, counts, b1.reshape(1, H), W2)
    acc2 = _prop_o(g2, edges, pads, zeros_o)
    logits = _final(acc2, g2, counts, b2.reshape(1, O))
    return h, logits


# speedup vs baseline: 1.2552x; 1.0011x over previous
import functools

import jax
import jax.numpy as jnp
from jax import lax
from jax.experimental import pallas as pl
from jax.experimental.pallas import tpu as pltpu
from jax.experimental.pallas import tpu_sc as plsc

N = 10000
E = 320000
D = 128
H = 128
O = 64

NC = 2
NS = 16
NW = NC * NS
CE = 128
NCH = 80
RE = E // CE
TAIL = RE - (NW - 1) * NCH
PAD_CH = NW * NCH - RE
NPAD = 10240
RPS = NPAD // NS

_mesh = plsc.VectorSubcoreMesh(core_axis_name="c", subcore_axis_name="s")


@functools.partial(
    pl.kernel,
    out_type=jax.ShapeDtypeStruct((NC, NPAD), jnp.float32),
    mesh=_mesh,
    scratch_types=[
        pltpu.VMEM((NCH * CE,), jnp.int32),
        pltpu.VMEM((CE,), jnp.float32),
        pltpu.VMEM_SHARED((NPAD,), jnp.float32),
        pltpu.SemaphoreType.DMA,
    ],
)
def _degree(edge_hbm, pads_hbm, ones_hbm, zeros1_hbm, out_hbm,
            dst_v, ones_v, cnt_sh, sem):
    c = lax.axis_index("c")
    s = lax.axis_index("s")
    wid = s * NC + c

    @pl.when(wid < NW - 1)
    def _():
        pltpu.async_copy(edge_hbm.at[1].at[pl.ds(wid * (NCH * CE), NCH * CE)],
                         dst_v, sem).wait()

    @pl.when(wid == NW - 1)
    def _():
        cp_a = pltpu.async_copy(
            edge_hbm.at[1].at[pl.ds((RE - TAIL) * CE, TAIL * CE)],
            dst_v.at[pl.ds(0, TAIL * CE)], sem)
        cp_b = pltpu.async_copy(pads_hbm,
                                dst_v.at[pl.ds(TAIL * CE, PAD_CH * CE)], sem)
        cp_a.wait()
        cp_b.wait()

    pltpu.async_copy(ones_hbm, ones_v, sem).wait()
    pltpu.async_copy(
        zeros1_hbm.at[pl.ds(s * RPS, RPS)], cnt_sh.at[pl.ds(s * RPS, RPS)], sem
    ).wait()
    plsc.subcore_barrier()

    @pl.loop(0, NCH)
    def _(j):
        pltpu.sync_copy(ones_v, cnt_sh.at[dst_v.at[pl.ds(j * CE, CE)]], add=True)

    plsc.subcore_barrier()
    pltpu.async_copy(
        cnt_sh.at[pl.ds(s * RPS, RPS)], out_hbm.at[c].at[pl.ds(s * RPS, RPS)], sem
    ).wait()


def _make_propagate(width):
    cp = (pltpu.CompilerParams(use_tc_tiling_on_sc=False)
          if width != H else pltpu.CompilerParams())
    @functools.partial(
        pl.kernel,
        out_type=jax.ShapeDtypeStruct((NC, NPAD, width), jnp.float32),
        mesh=_mesh,
        compiler_params=cp,
        scratch_types=[
            pltpu.VMEM((NCH // 2 * CE,), jnp.int32),
            pltpu.VMEM((NCH // 2 * CE,), jnp.int32),
            pltpu.VMEM((CE, width), jnp.float32),
            pltpu.VMEM((CE, width), jnp.float32),
            pltpu.VMEM_SHARED((NPAD, width), jnp.float32),
            pltpu.SemaphoreType.DMA,
            pltpu.SemaphoreType.DMA,
            pltpu.SemaphoreType.DMA,
        ],
    )
    def prop(g_hbm, edge_hbm, pads_hbm, zeros_hbm, out_hbm,
             src_v, dst_v, buf0, buf1, acc_sh, sem, gsem0, gsem1):
        c = lax.axis_index("c")
        s = lax.axis_index("s")
        wid = s * NC + c
        hch = NCH // 2
        pltpu.async_copy(
            zeros_hbm.at[pl.ds(s * RPS, RPS)], acc_sh.at[pl.ds(s * RPS, RPS)], sem
        ).wait()
        plsc.subcore_barrier()

        @pl.loop(0, 2)
        def _(p):
            e0 = (wid * NCH + p * hch) * CE

            @pl.when(wid < NW - 1)
            def _():
                cp_s = pltpu.async_copy(edge_hbm.at[0].at[pl.ds(e0, hch * CE)],
                                        src_v, sem)
                cp_d = pltpu.async_copy(edge_hbm.at[1].at[pl.ds(e0, hch * CE)],
                                        dst_v, gsem0)
                cp_s.wait()
                cp_d.wait()

            @pl.when((wid == NW - 1) & (p == 0))
            def _():
                cp_a = pltpu.async_copy(
                    edge_hbm.at[0].at[pl.ds((RE - TAIL) * CE, TAIL * CE)],
                    src_v.at[pl.ds(0, TAIL * CE)], sem)
                cp_b = pltpu.async_copy(
                    pads_hbm.at[pl.ds(0, (hch - TAIL) * CE)],
                    src_v.at[pl.ds(TAIL * CE, (hch - TAIL) * CE)], gsem0)
                cp_c = pltpu.async_copy(
                    edge_hbm.at[1].at[pl.ds((RE - TAIL) * CE, TAIL * CE)],
                    dst_v.at[pl.ds(0, TAIL * CE)], gsem1)
                cp_d = pltpu.async_copy(
                    pads_hbm.at[pl.ds(0, (hch - TAIL) * CE)],
                    dst_v.at[pl.ds(TAIL * CE, (hch - TAIL) * CE)], sem)
                cp_a.wait()
                cp_b.wait()
                cp_c.wait()
                cp_d.wait()

            @pl.when((wid == NW - 1) & (p == 1))
            def _():
                cp_s = pltpu.async_copy(
                    pads_hbm.at[pl.ds((PAD_CH - hch) * CE, hch * CE)], src_v, sem)
                cp_d = pltpu.async_copy(
                    pads_hbm.at[pl.ds((PAD_CH - hch) * CE, hch * CE)], dst_v, gsem0)
                cp_s.wait()
                cp_d.wait()
            pltpu.async_copy(g_hbm.at[src_v.at[pl.ds(0, CE)]], buf0, gsem0)
            pltpu.async_copy(g_hbm.at[src_v.at[pl.ds(CE, CE)]], buf1, gsem1)

            @pl.loop(0, hch, step=2)
            def _(j):
                pltpu.make_async_copy(
                    g_hbm.at[src_v.at[pl.ds(j * CE, CE)]], buf0, gsem0).wait()
                pltpu.sync_copy(buf0, acc_sh.at[dst_v.at[pl.ds(j * CE, CE)]],
                                add=True)

                @pl.when(j + 2 < hch)
                def _():
                    pltpu.async_copy(
                        g_hbm.at[src_v.at[pl.ds((j + 2) * CE, CE)]], buf0, gsem0)

                pltpu.make_async_copy(
                    g_hbm.at[src_v.at[pl.ds((j + 1) * CE, CE)]], buf1, gsem1).wait()
                pltpu.sync_copy(buf1, acc_sh.at[dst_v.at[pl.ds((j + 1) * CE, CE)]],
                                add=True)

                @pl.when(j + 3 < hch)
                def _():
                    pltpu.async_copy(
                        g_hbm.at[src_v.at[pl.ds((j + 3) * CE, CE)]], buf1, gsem1)

        plsc.subcore_barrier()
        pltpu.async_copy(
            acc_sh.at[pl.ds(s * RPS, RPS)],
            out_hbm.at[c].at[pl.ds(s * RPS, RPS)],
            sem,
        ).wait()

    return prop


_prop_h = _make_propagate(H)
_prop_o = _make_propagate(O)


_BLK = 1280
_GRID = NPAD // _BLK


def _matmul(x, w):
    d_in, d_out = w.shape

    def body(x_ref, w_ref, o_ref):
        o_ref[...] = jnp.dot(x_ref[...], w_ref[...],
                             preferred_element_type=jnp.float32)

    return pl.pallas_call(
        body,
        grid=(_GRID,),
        in_specs=[
            pl.BlockSpec((_BLK, d_in), lambda i: (i, 0)),
            pl.BlockSpec((d_in, d_out), lambda i: (0, 0)),
        ],
        out_specs=pl.BlockSpec((_BLK, d_out), lambda i: (i, 0)),
        out_shape=jax.ShapeDtypeStruct((NPAD, d_out), jnp.float32),
    )(x, w)


def _scale(h, counts):
    def body(h_ref, c_ref, o_ref):
        dinv = lax.rsqrt(1.0 + c_ref[0, :] + c_ref[1, :])
        o_ref[...] = h_ref[...] * dinv[:, None]

    return pl.pallas_call(
        body,
        grid=(_GRID,),
        in_specs=[
            pl.BlockSpec((_BLK, H), lambda i: (i, 0)),
            pl.BlockSpec((NC, _BLK), lambda i: (0, i)),
        ],
        out_specs=pl.BlockSpec((_BLK, H), lambda i: (i, 0)),
        out_shape=jax.ShapeDtypeStruct((NPAD, H), jnp.float32),
    )(h, counts)


def _layer2(acc1, g1, counts, b1, w2):
    def body(a_ref, g_ref, c_ref, b_ref, w_ref, h_ref, g2_ref):
        dinv = lax.rsqrt(1.0 + c_ref[0, :] + c_ref[1, :])
        pre = (a_ref[0] + a_ref[1] + g_ref[...]) * dinv[:, None] + b_ref[...]
        hblk = jnp.maximum(pre, 0.0)
        h_ref[...] = hblk
        g2_ref[...] = jnp.dot(hblk, w_ref[...],
                              preferred_element_type=jnp.float32) * dinv[:, None]

    return pl.pallas_call(
        body,
        grid=(_GRID,),
        in_specs=[
            pl.BlockSpec((NC, _BLK, H), lambda i: (0, i, 0)),
            pl.BlockSpec((_BLK, H), lambda i: (i, 0)),
            pl.BlockSpec((NC, _BLK), lambda i: (0, i)),
            pl.BlockSpec((1, H), lambda i: (0, 0)),
            pl.BlockSpec((H, O), lambda i: (0, 0)),
        ],
        out_specs=[
            pl.BlockSpec((_BLK, H), lambda i: (i, 0)),
            pl.BlockSpec((_BLK, O), lambda i: (i, 0)),
        ],
        out_shape=[
            jax.ShapeDtypeStruct((N, H), jnp.float32),
            jax.ShapeDtypeStruct((NPAD, O), jnp.float32),
        ],
    )(acc1, g1, counts, b1, w2)


def _final(acc2, g2, counts, b2):
    def body(a_ref, g_ref, c_ref, b_ref, o_ref):
        dinv = lax.rsqrt(1.0 + c_ref[0, :] + c_ref[1, :])
        o_ref[...] = (a_ref[0] + a_ref[1] + g_ref[...]) * dinv[:, None] + b_ref[...]

    return pl.pallas_call(
        body,
        grid=(_GRID,),
        in_specs=[
            pl.BlockSpec((NC, _BLK, O), lambda i: (0, i, 0)),
            pl.BlockSpec((_BLK, O), lambda i: (i, 0)),
            pl.BlockSpec((NC, _BLK), lambda i: (0, i)),
            pl.BlockSpec((1, O), lambda i: (0, 0)),
        ],
        out_specs=pl.BlockSpec((_BLK, O), lambda i: (i, 0)),
        out_shape=jax.ShapeDtypeStruct((N, O), jnp.float32),
    )(acc2, g2, counts, b2)


def kernel(x, edge_index, W1, b1, W2, b2):
    edges = edge_index.astype(jnp.int32)
    pads = N + jnp.arange(PAD_CH * CE, dtype=jnp.int32) % (NPAD - N)
    ones_c = jnp.ones((CE,), jnp.float32)
    zeros1 = jnp.zeros((NPAD,), jnp.float32)
    zeros_h = jnp.zeros((NPAD, H), jnp.float32)
    zeros_o = jnp.zeros((NPAD, O), jnp.float32)

    counts = _degree(edges, pads, ones_c, zeros1)
    h1 = _matmul(x, W1)
    g1 = _scale(h1, counts)
    acc1 = _prop_h(g1, edges, pads, zeros_h)
    h, g2 = _layer2(acc1, g1, counts, b1.reshape(1, H), W2)
    acc2 = _prop_o(g2, edges, pads, zeros_o)
    logits = _final(acc2, g2, counts, b2.reshape(1, O))
    return h, logits


# speedup vs baseline: 1.2560x; 1.0006x over previous
import functools

import jax
import jax.numpy as jnp
from jax import lax
from jax.experimental import pallas as pl
from jax.experimental.pallas import tpu as pltpu
from jax.experimental.pallas import tpu_sc as plsc

N = 10000
E = 320000
D = 128
H = 128
O = 64

NC = 2
NS = 16
NW = NC * NS
CE = 128
NCH = 80
RE = E // CE
TAIL = RE - (NW - 1) * NCH
PAD_CH = NW * NCH - RE
NPAD = 10240
RPS = NPAD // NS

_mesh = plsc.VectorSubcoreMesh(core_axis_name="c", subcore_axis_name="s")


@functools.partial(
    pl.kernel,
    out_type=jax.ShapeDtypeStruct((NC, NPAD), jnp.float32),
    mesh=_mesh,
    scratch_types=[
        pltpu.VMEM((NCH * CE,), jnp.int32),
        pltpu.VMEM((CE,), jnp.float32),
        pltpu.VMEM_SHARED((NPAD,), jnp.float32),
        pltpu.SemaphoreType.DMA,
    ],
)
def _degree(edge_hbm, pads_hbm, ones_hbm, zeros1_hbm, out_hbm,
            dst_v, ones_v, cnt_sh, sem):
    c = lax.axis_index("c")
    s = lax.axis_index("s")
    wid = s * NC + c

    @pl.when(wid < NW - 1)
    def _():
        pltpu.async_copy(edge_hbm.at[1].at[pl.ds(wid * (NCH * CE), NCH * CE)],
                         dst_v, sem).wait()

    @pl.when(wid == NW - 1)
    def _():
        cp_a = pltpu.async_copy(
            edge_hbm.at[1].at[pl.ds((RE - TAIL) * CE, TAIL * CE)],
            dst_v.at[pl.ds(0, TAIL * CE)], sem)
        cp_b = pltpu.async_copy(pads_hbm,
                                dst_v.at[pl.ds(TAIL * CE, PAD_CH * CE)], sem)
        cp_a.wait()
        cp_b.wait()

    pltpu.async_copy(ones_hbm, ones_v, sem).wait()
    pltpu.async_copy(
        zeros1_hbm.at[pl.ds(s * RPS, RPS)], cnt_sh.at[pl.ds(s * RPS, RPS)], sem
    ).wait()
    plsc.subcore_barrier()

    @pl.loop(0, NCH)
    def _(j):
        pltpu.sync_copy(ones_v, cnt_sh.at[dst_v.at[pl.ds(j * CE, CE)]], add=True)

    plsc.subcore_barrier()
    pltpu.async_copy(
        cnt_sh.at[pl.ds(s * RPS, RPS)], out_hbm.at[c].at[pl.ds(s * RPS, RPS)], sem
    ).wait()


def _make_propagate(width):
    @functools.partial(
        pl.kernel,
        out_type=jax.ShapeDtypeStruct((NC, NPAD, width), jnp.float32),
        mesh=_mesh,
        compiler_params=pltpu.CompilerParams(use_tc_tiling_on_sc=False),
        scratch_types=[
            pltpu.VMEM((NCH // 2 * CE,), jnp.int32),
            pltpu.VMEM((NCH // 2 * CE,), jnp.int32),
            pltpu.VMEM((CE, width), jnp.float32),
            pltpu.VMEM((CE, width), jnp.float32),
            pltpu.VMEM_SHARED((NPAD, width), jnp.float32),
            pltpu.SemaphoreType.DMA,
            pltpu.SemaphoreType.DMA,
            pltpu.SemaphoreType.DMA,
        ],
    )
    def prop(g_hbm, edge_hbm, pads_hbm, zeros_hbm, out_hbm,
             src_v, dst_v, buf0, buf1, acc_sh, sem, gsem0, gsem1):
        c = lax.axis_index("c")
        s = lax.axis_index("s")
        wid = s * NC + c
        hch = NCH // 2
        pltpu.async_copy(
            zeros_hbm.at[pl.ds(s * RPS, RPS)], acc_sh.at[pl.ds(s * RPS, RPS)], sem
        ).wait()
        plsc.subcore_barrier()

        @pl.loop(0, 2)
        def _(p):
            e0 = (wid * NCH + p * hch) * CE

            @pl.when(wid < NW - 1)
            def _():
                cp_s = pltpu.async_copy(edge_hbm.at[0].at[pl.ds(e0, hch * CE)],
                                        src_v, sem)
                cp_d = pltpu.async_copy(edge_hbm.at[1].at[pl.ds(e0, hch * CE)],
                                        dst_v, gsem0)
                cp_s.wait()
                cp_d.wait()

            @pl.when((wid == NW - 1) & (p == 0))
            def _():
                cp_a = pltpu.async_copy(
                    edge_hbm.at[0].at[pl.ds((RE - TAIL) * CE, TAIL * CE)],
                    src_v.at[pl.ds(0, TAIL * CE)], sem)
                cp_b = pltpu.async_copy(
                    pads_hbm.at[pl.ds(0, (hch - TAIL) * CE)],
                    src_v.at[pl.ds(TAIL * CE, (hch - TAIL) * CE)], gsem0)
                cp_c = pltpu.async_copy(
                    edge_hbm.at[1].at[pl.ds((RE - TAIL) * CE, TAIL * CE)],
                    dst_v.at[pl.ds(0, TAIL * CE)], gsem1)
                cp_d = pltpu.async_copy(
                    pads_hbm.at[pl.ds(0, (hch - TAIL) * CE)],
                    dst_v.at[pl.ds(TAIL * CE, (hch - TAIL) * CE)], sem)
                cp_a.wait()
                cp_b.wait()
                cp_c.wait()
                cp_d.wait()

            @pl.when((wid == NW - 1) & (p == 1))
            def _():
                cp_s = pltpu.async_copy(
                    pads_hbm.at[pl.ds((PAD_CH - hch) * CE, hch * CE)], src_v, sem)
                cp_d = pltpu.async_copy(
                    pads_hbm.at[pl.ds((PAD_CH - hch) * CE, hch * CE)], dst_v, gsem0)
                cp_s.wait()
                cp_d.wait()
            pltpu.async_copy(g_hbm.at[src_v.at[pl.ds(0, CE)]], buf0, gsem0)
            pltpu.async_copy(g_hbm.at[src_v.at[pl.ds(CE, CE)]], buf1, gsem1)

            @pl.loop(0, hch, step=2)
            def _(j):
                pltpu.make_async_copy(
                    g_hbm.at[src_v.at[pl.ds(j * CE, CE)]], buf0, gsem0).wait()
                pltpu.sync_copy(buf0, acc_sh.at[dst_v.at[pl.ds(j * CE, CE)]],
                                add=True)

                @pl.when(j + 2 < hch)
                def _():
                    pltpu.async_copy(
                        g_hbm.at[src_v.at[pl.ds((j + 2) * CE, CE)]], buf0, gsem0)

                pltpu.make_async_copy(
                    g_hbm.at[src_v.at[pl.ds((j + 1) * CE, CE)]], buf1, gsem1).wait()
                pltpu.sync_copy(buf1, acc_sh.at[dst_v.at[pl.ds((j + 1) * CE, CE)]],
                                add=True)

                @pl.when(j + 3 < hch)
                def _():
                    pltpu.async_copy(
                        g_hbm.at[src_v.at[pl.ds((j + 3) * CE, CE)]], buf1, gsem1)

        plsc.subcore_barrier()
        pltpu.async_copy(
            acc_sh.at[pl.ds(s * RPS, RPS)],
            out_hbm.at[c].at[pl.ds(s * RPS, RPS)],
            sem,
        ).wait()

    return prop


_prop_h = _make_propagate(H)
_prop_o = _make_propagate(O)


_BLK = 1280
_GRID = NPAD // _BLK


def _matmul(x, w):
    d_in, d_out = w.shape

    def body(x_ref, w_ref, o_ref):
        o_ref[...] = jnp.dot(x_ref[...], w_ref[...],
                             preferred_element_type=jnp.float32)

    return pl.pallas_call(
        body,
        grid=(_GRID,),
        in_specs=[
            pl.BlockSpec((_BLK, d_in), lambda i: (i, 0)),
            pl.BlockSpec((d_in, d_out), lambda i: (0, 0)),
        ],
        out_specs=pl.BlockSpec((_BLK, d_out), lambda i: (i, 0)),
        out_shape=jax.ShapeDtypeStruct((NPAD, d_out), jnp.float32),
    )(x, w)


def _scale(h, counts):
    def body(h_ref, c_ref, o_ref):
        dinv = lax.rsqrt(1.0 + c_ref[0, :] + c_ref[1, :])
        o_ref[...] = h_ref[...] * dinv[:, None]

    return pl.pallas_call(
        body,
        grid=(_GRID,),
        in_specs=[
            pl.BlockSpec((_BLK, H), lambda i: (i, 0)),
            pl.BlockSpec((NC, _BLK), lambda i: (0, i)),
        ],
        out_specs=pl.BlockSpec((_BLK, H), lambda i: (i, 0)),
        out_shape=jax.ShapeDtypeStruct((NPAD, H), jnp.float32),
    )(h, counts)


def _layer2(acc1, g1, counts, b1, w2):
    def body(a_ref, g_ref, c_ref, b_ref, w_ref, h_ref, g2_ref):
        dinv = lax.rsqrt(1.0 + c_ref[0, :] + c_ref[1, :])
        pre = (a_ref[0] + a_ref[1] + g_ref[...]) * dinv[:, None] + b_ref[...]
        hblk = jnp.maximum(pre, 0.0)
        h_ref[...] = hblk
        g2_ref[...] = jnp.dot(hblk, w_ref[...],
                              preferred_element_type=jnp.float32) * dinv[:, None]

    return pl.pallas_call(
        body,
        grid=(_GRID,),
        in_specs=[
            pl.BlockSpec((NC, _BLK, H), lambda i: (0, i, 0)),
            pl.BlockSpec((_BLK, H), lambda i: (i, 0)),
            pl.BlockSpec((NC, _BLK), lambda i: (0, i)),
            pl.BlockSpec((1, H), lambda i: (0, 0)),
            pl.BlockSpec((H, O), lambda i: (0, 0)),
        ],
        out_specs=[
            pl.BlockSpec((_BLK, H), lambda i: (i, 0)),
            pl.BlockSpec((_BLK, O), lambda i: (i, 0)),
        ],
        out_shape=[
            jax.ShapeDtypeStruct((N, H), jnp.float32),
            jax.ShapeDtypeStruct((NPAD, O), jnp.float32),
        ],
    )(acc1, g1, counts, b1, w2)


def _final(acc2, g2, counts, b2):
    def body(a_ref, g_ref, c_ref, b_ref, o_ref):
        dinv = lax.rsqrt(1.0 + c_ref[0, :] + c_ref[1, :])
        o_ref[...] = (a_ref[0] + a_ref[1] + g_ref[...]) * dinv[:, None] + b_ref[...]

    return pl.pallas_call(
        body,
        grid=(_GRID,),
        in_specs=[
            pl.BlockSpec((NC, _BLK, O), lambda i: (0, i, 0)),
            pl.BlockSpec((_BLK, O), lambda i: (i, 0)),
            pl.BlockSpec((NC, _BLK), lambda i: (0, i)),
            pl.BlockSpec((1, O), lambda i: (0, 0)),
        ],
        out_specs=pl.BlockSpec((_BLK, O), lambda i: (i, 0)),
        out_shape=jax.ShapeDtypeStruct((N, O), jnp.float32),
    )(acc2, g2, counts, b2)


def kernel(x, edge_index, W1, b1, W2, b2):
    edges = edge_index.astype(jnp.int32)
    pads = N + jnp.arange(PAD_CH * CE, dtype=jnp.int32) % (NPAD - N)
    ones_c = jnp.ones((CE,), jnp.float32)
    zeros1 = jnp.zeros((NPAD,), jnp.float32)
    zeros_h = jnp.zeros((NPAD, H), jnp.float32)
    zeros_o = jnp.zeros((NPAD, O), jnp.float32)

    counts = _degree(edges, pads, ones_c, zeros1)
    h1 = _matmul(x, W1)
    g1 = _scale(h1, counts)
    acc1 = _prop_h(g1, edges, pads, zeros_h)
    h, g2 = _layer2(acc1, g1, counts, b1.reshape(1, H), W2)
    acc2 = _prop_o(g2, edges, pads, zeros_o)
    logits = _final(acc2, g2, counts, b2.reshape(1, O))
    return h, logits
